# baseline (speedup 1.0000x reference)
import functools

import jax
import jax.numpy as jnp
import numpy as np
from jax import lax
from jax.experimental import pallas as pl
from jax.experimental.pallas import tpu as pltpu
from jax.experimental.pallas import tpu_sc as plsc

OUT = 7
SR = 2
NTAP = 784
NTAP_PAD = 896
CHUNK = 112
NCHUNK = 7
C = 256
R_TOTAL = 1024
ROIS_PER_SUBCORE = 32
OUTW = C * 49

_j = np.arange(NTAP_PAD)
_bin = _j // 16
_q = _j % 16
_si = _q // 4
_corner = _q % 4
_dy = (_corner // 2).astype(np.float32)
_dx = (_corner % 2).astype(np.float32)
_sy = np.minimum(_bin // 7, 6) * 2 + _si // 2
_sx = np.minimum(_bin % 7, 6) * 2 + _si % 2
_grid = (np.arange(OUT, dtype=np.float32)[:, None]
         + (np.arange(SR, dtype=np.float32)[None, :] + 0.5) / SR).reshape(-1)
_gy = _grid[_sy].astype(np.float32)
_gx = _grid[_sx].astype(np.float32)

_LVL_W = (256.0, 128.0, 64.0, 32.0)
_LVL_SCALE = (0.25, 0.125, 0.0625, 0.03125)
_LVL_BASE = (0.0, 131072.0, 163840.0, 172032.0)


def _tap_kernel(bp_ref, gy_ref, gx_ref, dy_ref, dx_ref, idx_ref, w_ref):
    bp = bp_ref[:]
    x0b = bp[:, 0:1]
    y0b = bp[:, 1:2]
    x1b = bp[:, 2:3]
    y1b = bp[:, 3:4]
    lv = bp[:, 4:5]
    bimg = bp[:, 5:6]

    def lvsel(vals):
        return jnp.where(
            lv < 0.5, vals[0],
            jnp.where(lv < 1.5, vals[1], jnp.where(lv < 2.5, vals[2], vals[3])))

    scale = lvsel(_LVL_SCALE)
    Wf = lvsel(_LVL_W)
    Hf = Wf
    base = lvsel(_LVL_BASE) + bimg * Hf * Wf

    x0 = x0b * scale - 0.5
    y0 = y0b * scale - 0.5
    x1 = x1b * scale - 0.5
    y1 = y1b * scale - 0.5
    bw = (x1 - x0) / OUT
    bh = (y1 - y0) / OUT

    gy = gy_ref[:]
    gx = gx_ref[:]
    dy = dy_ref[:]
    dx = dx_ref[:]

    ys = y0 + bh * gy
    xs = x0 + bw * gx
    vy = ((ys >= -1.0) & (ys <= Hf)).astype(jnp.float32)
    vx = ((xs >= -1.0) & (xs <= Wf)).astype(jnp.float32)
    yc = jnp.clip(ys, 0.0, Hf - 1.0)
    xc = jnp.clip(xs, 0.0, Wf - 1.0)
    yl = jnp.floor(yc)
    xl = jnp.floor(xc)
    ly = yc - yl
    lx = xc - xl
    wy = dy * ly + (1.0 - dy) * (1.0 - ly)
    wx = dx * lx + (1.0 - dx) * (1.0 - lx)
    yi = jnp.minimum(yl + dy, Hf - 1.0)
    xi = jnp.minimum(xl + dx, Wf - 1.0)

    idx_ref[:] = (base + yi * Wf + xi).astype(jnp.int32)
    w_ref[:] = wy * wx * vy * vx * 0.25


def _compute_taps(bp):
    BR = 256
    grid = (R_TOTAL // BR,)
    consts = [jnp.asarray(a).reshape(1, NTAP_PAD) for a in (_gy, _gx, _dy, _dx)]
    idx, w = pl.pallas_call(
        _tap_kernel,
        grid=grid,
        in_specs=[
            pl.BlockSpec((BR, 8), lambda i: (i, 0)),
            pl.BlockSpec((1, NTAP_PAD), lambda i: (0, 0)),
            pl.BlockSpec((1, NTAP_PAD), lambda i: (0, 0)),
            pl.BlockSpec((1, NTAP_PAD), lambda i: (0, 0)),
            pl.BlockSpec((1, NTAP_PAD), lambda i: (0, 0)),
        ],
        out_specs=[
            pl.BlockSpec((BR, NTAP_PAD), lambda i: (i, 0)),
            pl.BlockSpec((BR, NTAP_PAD), lambda i: (i, 0)),
        ],
        out_shape=[
            jax.ShapeDtypeStruct((R_TOTAL, NTAP_PAD), jnp.int32),
            jax.ShapeDtypeStruct((R_TOTAL, NTAP_PAD), jnp.float32),
        ],
    )(bp, *consts)
    return idx, w


def _pack_level(f):
    n, _, h, w2 = f.shape
    u = lax.bitcast_convert_type(f, jnp.uint32)
    ur = u.reshape(n, 8, 2, 16, h, w2)
    ua, ub = ur[:, :, 0], ur[:, :, 1]
    ra = (ua + jnp.uint32(0x7FFF) + ((ua >> 16) & jnp.uint32(1))) >> 16
    rb = ((ub + jnp.uint32(0x7FFF) + ((ub >> 16) & jnp.uint32(1)))
          & jnp.uint32(0xFFFF0000))
    pk = (ra | rb).reshape(n, 128, h, w2)
    return pk.transpose(0, 2, 3, 1).reshape(-1, C // 2)


def _sc_body(table, idxm, wm, out_hbm,
             idx_v, w_v, rows0, rows1, outbuf, sem0, sem1, semo):
    wid = lax.axis_index("s") * 2 + lax.axis_index("c")
    GROUP = 16
    blk = GROUP * NTAP_PAD

    def group_body(g, carry0):
        goff = pl.multiple_of((wid * ROIS_PER_SUBCORE + g * GROUP) * NTAP_PAD, 8)
        pltpu.sync_copy(idxm.at[pl.ds(goff, blk)], idx_v)
        pltpu.sync_copy(wm.at[pl.ds(goff, blk)], w_v)

        def roi_body(k, carry):
            r = wid * ROIS_PER_SUBCORE + g * GROUP + k
            kbase = pl.multiple_of(k * NTAP_PAD, 8)

            bufs = (rows0, rows1)
            sems = (sem0, sem1)

            def start_gather(ci):
                return pltpu.async_copy(
                    table.at[idx_v.at[pl.ds(kbase + ci * CHUNK, CHUNK)]],
                    bufs[ci % 2], sems[ci % 2])

            cp = start_gather(0)
            for chunk in range(NCHUNK):
                cp.wait()
                if chunk + 1 < NCHUNK:
                    cp = start_gather(chunk + 1)
                rows = bufs[chunk % 2]

                def bin_body(b, _):
                    tap0 = b * 16
                    jbase = pl.multiple_of(kbase + chunk * CHUNK + tap0, 8)
                    wrow = w_v[pl.ds(jbase, 16)]
                    accs = [jnp.zeros((16,), jnp.float32) for _ in range(16)]
                    for t in range(16):
                        wsp = wrow[t]
                        for cc8 in range(8):
                            u = rows[tap0 + t, pl.ds(cc8 * 16, 16)]
                            lo = plsc.bitcast(u << 16, jnp.float32)
                            hi = plsc.bitcast(u, jnp.float32)
                            accs[2 * cc8] = accs[2 * cc8] + wsp * lo
                            accs[2 * cc8 + 1] = accs[2 * cc8 + 1] + wsp * hi
                    bing = chunk * 7 + b
                    lanes49 = lax.iota(jnp.int32, 16) * 49 + bing
                    for cc in range(16):
                        plsc.store_scatter(outbuf, [lanes49 + cc * 784],
                                           accs[cc])
                    return 0

                lax.fori_loop(0, 7, bin_body, 0)
            out_off = pl.multiple_of(r * OUTW, 8)
            pltpu.async_copy(outbuf, out_hbm.at[pl.ds(out_off, OUTW)],
                             semo).wait()
            return carry

        lax.fori_loop(0, GROUP, roi_body, 0)
        return carry0

    lax.fori_loop(0, ROIS_PER_SUBCORE // GROUP, group_body, 0)


def _sc_pool(table, idxm, wm):
    mesh = plsc.VectorSubcoreMesh(core_axis_name="c", subcore_axis_name="s")
    f = pl.kernel(
        _sc_body,
        out_type=jax.ShapeDtypeStruct((R_TOTAL * OUTW,), jnp.float32),
        mesh=mesh,
        compiler_params=pltpu.CompilerParams(needs_layout_passes=False),
        scratch_types=[
            pltpu.VMEM((16 * NTAP_PAD,), jnp.int32),
            pltpu.VMEM((16 * NTAP_PAD,), jnp.float32),
            pltpu.VMEM((CHUNK, C // 2), jnp.uint32),
            pltpu.VMEM((CHUNK, C // 2), jnp.uint32),
            pltpu.VMEM((OUTW,), jnp.float32),
            pltpu.SemaphoreType.DMA,
            pltpu.SemaphoreType.DMA,
            pltpu.SemaphoreType.DMA,
        ],
    )
    return f(table, idxm, wm)


def kernel(feat_p2, feat_p3, feat_p4, feat_p5, boxes):
    n_img, n_box = boxes.shape[0], boxes.shape[1]
    flat = boxes.reshape(-1, 4)

    sizes = jnp.clip(flat[:, 2:] - flat[:, :2], 0.0, None)
    areas = sizes[:, 0] * sizes[:, 1]
    s = jnp.sqrt(areas)
    lvf = jnp.floor(4.0 + jnp.log2(s / 224.0 + 1e-8))
    lv = jnp.clip(lvf, 2.0, 5.0) - 2.0

    bimg = jnp.repeat(jnp.arange(n_img, dtype=jnp.float32), n_box)
    pad = jnp.zeros((flat.shape[0], 2), jnp.float32)
    bp = jnp.concatenate([flat, lv[:, None], bimg[:, None], pad], axis=1)

    idx, w = _compute_taps(bp)

    table = jnp.concatenate([
        _pack_level(f) for f in (feat_p2, feat_p3, feat_p4, feat_p5)
    ], axis=0)

    out = _sc_pool(table, idx.reshape(-1), w.reshape(-1))
    return out.reshape(R_TOTAL, C, OUT, OUT)


# speedup vs baseline: 1.8622x; 1.8622x over previous
import functools

import jax
import jax.numpy as jnp
import numpy as np
from jax import lax
from jax.experimental import pallas as pl
from jax.experimental.pallas import tpu as pltpu
from jax.experimental.pallas import tpu_sc as plsc

OUT = 7
SR = 2
NTAP = 784
NTAP_PAD = 896
CHUNK = 112
NCHUNK = 7
C = 256
R_TOTAL = 1024
ROIS_PER_SUBCORE = 32
OUTW = C * 49

_j = np.arange(NTAP_PAD)
_bin = _j // 16
_q = _j % 16
_si = _q // 4
_corner = _q % 4
_dy = (_corner // 2).astype(np.float32)
_dx = (_corner % 2).astype(np.float32)
_sy = np.minimum(_bin // 7, 6) * 2 + _si // 2
_sx = np.minimum(_bin % 7, 6) * 2 + _si % 2
_grid = (np.arange(OUT, dtype=np.float32)[:, None]
         + (np.arange(SR, dtype=np.float32)[None, :] + 0.5) / SR).reshape(-1)
_gy = _grid[_sy].astype(np.float32)
_gx = _grid[_sx].astype(np.float32)

_LVL_W = (256.0, 128.0, 64.0, 32.0)
_LVL_SCALE = (0.25, 0.125, 0.0625, 0.03125)
_LVL_BASE = (0.0, 131072.0, 163840.0, 172032.0)


def _tap_kernel(bp_ref, gy_ref, gx_ref, dy_ref, dx_ref, idx_ref, w_ref):
    bp = bp_ref[:]
    x0b = bp[:, 0:1]
    y0b = bp[:, 1:2]
    x1b = bp[:, 2:3]
    y1b = bp[:, 3:4]
    lv = bp[:, 4:5]
    bimg = bp[:, 5:6]

    def lvsel(vals):
        return jnp.where(
            lv < 0.5, vals[0],
            jnp.where(lv < 1.5, vals[1], jnp.where(lv < 2.5, vals[2], vals[3])))

    scale = lvsel(_LVL_SCALE)
    Wf = lvsel(_LVL_W)
    Hf = Wf
    base = lvsel(_LVL_BASE) + bimg * Hf * Wf

    x0 = x0b * scale - 0.5
    y0 = y0b * scale - 0.5
    x1 = x1b * scale - 0.5
    y1 = y1b * scale - 0.5
    bw = (x1 - x0) / OUT
    bh = (y1 - y0) / OUT

    gy = gy_ref[:]
    gx = gx_ref[:]
    dy = dy_ref[:]
    dx = dx_ref[:]

    ys = y0 + bh * gy
    xs = x0 + bw * gx
    vy = ((ys >= -1.0) & (ys <= Hf)).astype(jnp.float32)
    vx = ((xs >= -1.0) & (xs <= Wf)).astype(jnp.float32)
    yc = jnp.clip(ys, 0.0, Hf - 1.0)
    xc = jnp.clip(xs, 0.0, Wf - 1.0)
    yl = jnp.floor(yc)
    xl = jnp.floor(xc)
    ly = yc - yl
    lx = xc - xl
    wy = dy * ly + (1.0 - dy) * (1.0 - ly)
    wx = dx * lx + (1.0 - dx) * (1.0 - lx)
    yi = jnp.minimum(yl + dy, Hf - 1.0)
    xi = jnp.minimum(xl + dx, Wf - 1.0)

    idx_ref[:] = (base + yi * Wf + xi).astype(jnp.int32)
    w_ref[:] = wy * wx * vy * vx * 0.25


def _compute_taps(bp):
    BR = 256
    grid = (R_TOTAL // BR,)
    consts = [jnp.asarray(a).reshape(1, NTAP_PAD) for a in (_gy, _gx, _dy, _dx)]
    idx, w = pl.pallas_call(
        _tap_kernel,
        grid=grid,
        in_specs=[
            pl.BlockSpec((BR, 8), lambda i: (i, 0)),
            pl.BlockSpec((1, NTAP_PAD), lambda i: (0, 0)),
            pl.BlockSpec((1, NTAP_PAD), lambda i: (0, 0)),
            pl.BlockSpec((1, NTAP_PAD), lambda i: (0, 0)),
            pl.BlockSpec((1, NTAP_PAD), lambda i: (0, 0)),
        ],
        out_specs=[
            pl.BlockSpec((BR, NTAP_PAD), lambda i: (i, 0)),
            pl.BlockSpec((BR, NTAP_PAD), lambda i: (i, 0)),
        ],
        out_shape=[
            jax.ShapeDtypeStruct((R_TOTAL, NTAP_PAD), jnp.int32),
            jax.ShapeDtypeStruct((R_TOTAL, NTAP_PAD), jnp.float32),
        ],
    )(bp, *consts)
    return idx, w


def _pack_level(f):
    n, _, h, w2 = f.shape
    f5 = f.reshape(n, 16, 16, h, w2)
    fa = lax.slice(f5, (0, 0, 0, 0, 0), (n, 15, 16, h, w2), (1, 2, 1, 1, 1))
    fb = lax.slice(f5, (0, 1, 0, 0, 0), (n, 16, 16, h, w2), (1, 2, 1, 1, 1))
    a16 = lax.bitcast_convert_type(fa.astype(jnp.bfloat16), jnp.uint16)
    b16 = lax.bitcast_convert_type(fb.astype(jnp.bfloat16), jnp.uint16)
    pk = a16.astype(jnp.uint32) | (b16.astype(jnp.uint32) << 16)
    pk = pk.reshape(n, 128, h, w2)
    return pk.transpose(0, 2, 3, 1).reshape(-1, C // 2)


def _sc_body(table, idxm, wm, out_hbm,
             idx_v, w_v, rows0, rows1, outbuf, sem0, sem1, semo):
    wid = lax.axis_index("s") * 2 + lax.axis_index("c")
    GROUP = 16
    blk = GROUP * NTAP_PAD

    def group_body(g, carry0):
        goff = pl.multiple_of((wid * ROIS_PER_SUBCORE + g * GROUP) * NTAP_PAD, 8)
        pltpu.sync_copy(idxm.at[pl.ds(goff, blk)], idx_v)
        pltpu.sync_copy(wm.at[pl.ds(goff, blk)], w_v)

        def roi_body(k, carry):
            r = wid * ROIS_PER_SUBCORE + g * GROUP + k
            kbase = pl.multiple_of(k * NTAP_PAD, 8)

            bufs = (rows0, rows1)
            sems = (sem0, sem1)

            def start_gather(ci):
                return pltpu.async_copy(
                    table.at[idx_v.at[pl.ds(kbase + ci * CHUNK, CHUNK)]],
                    bufs[ci % 2], sems[ci % 2])

            cp = start_gather(0)
            for chunk in range(NCHUNK):
                cp.wait()
                if chunk + 1 < NCHUNK:
                    cp = start_gather(chunk + 1)
                rows = bufs[chunk % 2]

                def bin_body(b, _):
                    tap0 = b * 16
                    jbase = pl.multiple_of(kbase + chunk * CHUNK + tap0, 8)
                    wrow = w_v[pl.ds(jbase, 16)]
                    accs = [jnp.zeros((16,), jnp.float32) for _ in range(16)]
                    for t in range(16):
                        wsp = wrow[t]
                        for cc8 in range(8):
                            u = rows[tap0 + t, pl.ds(cc8 * 16, 16)]
                            lo = plsc.bitcast(u << 16, jnp.float32)
                            hi = plsc.bitcast(u, jnp.float32)
                            accs[2 * cc8] = accs[2 * cc8] + wsp * lo
                            accs[2 * cc8 + 1] = accs[2 * cc8 + 1] + wsp * hi
                    obase = pl.multiple_of((chunk * 7 + b) * C, 8)
                    for cc in range(16):
                        outbuf[pl.ds(obase + cc * 16, 16)] = accs[cc]
                    return 0

                lax.fori_loop(0, 7, bin_body, 0)
            out_off = pl.multiple_of(r * OUTW, 8)
            pltpu.async_copy(outbuf, out_hbm.at[pl.ds(out_off, OUTW)],
                             semo).wait()
            return carry

        lax.fori_loop(0, GROUP, roi_body, 0)
        return carry0

    lax.fori_loop(0, ROIS_PER_SUBCORE // GROUP, group_body, 0)


def _sc_pool(table, idxm, wm):
    mesh = plsc.VectorSubcoreMesh(core_axis_name="c", subcore_axis_name="s")
    f = pl.kernel(
        _sc_body,
        out_type=jax.ShapeDtypeStruct((R_TOTAL * OUTW,), jnp.float32),
        mesh=mesh,
        compiler_params=pltpu.CompilerParams(needs_layout_passes=False),
        scratch_types=[
            pltpu.VMEM((16 * NTAP_PAD,), jnp.int32),
            pltpu.VMEM((16 * NTAP_PAD,), jnp.float32),
            pltpu.VMEM((CHUNK, C // 2), jnp.uint32),
            pltpu.VMEM((CHUNK, C // 2), jnp.uint32),
            pltpu.VMEM((OUTW,), jnp.float32),
            pltpu.SemaphoreType.DMA,
            pltpu.SemaphoreType.DMA,
            pltpu.SemaphoreType.DMA,
        ],
    )
    return f(table, idxm, wm)


def kernel(feat_p2, feat_p3, feat_p4, feat_p5, boxes):
    n_img, n_box = boxes.shape[0], boxes.shape[1]
    flat = boxes.reshape(-1, 4)

    sizes = jnp.clip(flat[:, 2:] - flat[:, :2], 0.0, None)
    areas = sizes[:, 0] * sizes[:, 1]
    s = jnp.sqrt(areas)
    lvf = jnp.floor(4.0 + jnp.log2(s / 224.0 + 1e-8))
    lv = jnp.clip(lvf, 2.0, 5.0) - 2.0

    bimg = jnp.repeat(jnp.arange(n_img, dtype=jnp.float32), n_box)
    pad = jnp.zeros((flat.shape[0], 2), jnp.float32)
    bp = jnp.concatenate([flat, lv[:, None], bimg[:, None], pad], axis=1)

    idx, w = _compute_taps(bp)

    table = jnp.concatenate([
        _pack_level(f) for f in (feat_p2, feat_p3, feat_p4, feat_p5)
    ], axis=0)

    out = _sc_pool(table, idx.reshape(-1), w.reshape(-1))
    out = out.reshape(R_TOTAL, OUT * OUT, C).transpose(0, 2, 1)
    return out.reshape(R_TOTAL, C, OUT, OUT)
